# baseline (speedup 1.0000x reference)
import jax
import jax.numpy as jnp
from jax.experimental import pallas as pl
from jax.experimental.pallas import tpu as pltpu

B, S, H, D = 16, 2048, 8, 128
MAX_B, MAX_S = 16, 4096


def _copy_body(k_ref, v_ref, kc_ref, vc_ref, ok_ref, ov_ref,
               s0, s1, s2, s3):
    c0 = pltpu.make_async_copy(k_ref, ok_ref.at[:, 0:S], s0)
    c1 = pltpu.make_async_copy(kc_ref.at[:, S:MAX_S], ok_ref.at[:, S:MAX_S], s1)
    c2 = pltpu.make_async_copy(v_ref, ov_ref.at[:, 0:S], s2)
    c3 = pltpu.make_async_copy(vc_ref.at[:, S:MAX_S], ov_ref.at[:, S:MAX_S], s3)
    c0.start()
    c1.start()
    c2.start()
    c3.start()
    c0.wait()
    c1.wait()
    c2.wait()
    c3.wait()


def kernel(k, v, k_cache, v_cache):
    out_shape = jax.ShapeDtypeStruct((MAX_B, MAX_S, H, D), jnp.float32)
    hbm = pl.BlockSpec(memory_space=pltpu.MemorySpace.HBM)
    return pl.pallas_call(
        _copy_body,
        out_shape=(out_shape, out_shape),
        in_specs=[hbm, hbm, hbm, hbm],
        out_specs=(hbm, hbm),
        scratch_shapes=[pltpu.SemaphoreType.DMA] * 4,
    )(k, v, k_cache, v_cache)
